# baseline (speedup 1.0000x reference)
import functools

import jax
import jax.numpy as jnp
from jax import lax
from jax.experimental import pallas as pl
from jax.experimental.pallas import tpu as pltpu
from jax.experimental.pallas import tpu_sc as plsc

N = 10000
E = 320000
F = 128
NHALF = 2
EH = E // NHALF

NC = 2
NS = 16
NW = NC * NS

EW = EH // NW
C = 40
NCHUNK = EW // C
NUNIT = 2 * NCHUNK
NBUF = 5
assert NUNIT % NBUF == 0
NPAD = 10240
ROWS_PER_TILE = NPAD // NS

_mesh = plsc.VectorSubcoreMesh(core_axis_name="c", subcore_axis_name="s",
                               num_cores=NC, num_subcores=NS)


@functools.partial(
    pl.kernel,
    out_type=jax.ShapeDtypeStruct((2, EH, F), jnp.float32),
    mesh=_mesh,
    scratch_types=(
        [pltpu.VMEM((NUNIT, C), jnp.int32)]
        + [pltpu.VMEM((C, F), jnp.float32) for _ in range(NBUF)]
        + [pltpu.SemaphoreType.DMA for _ in range(2 * NBUF)]
    ),
)
def _sc_gather(x_hbm, idx2_hbm, out_hbm, idx_v, *bufs_and_sems):
    bufs = bufs_and_sems[:NBUF]
    gsem = bufs_and_sems[NBUF:2 * NBUF]
    wsem = bufs_and_sems[2 * NBUF:]
    wid = lax.axis_index("s") * NC + lax.axis_index("c")
    pltpu.sync_copy(idx2_hbm.at[wid], idx_v)

    def fire_g(u, p):
        pltpu.async_copy(x_hbm.at[idx_v.at[u]], bufs[p], gsem[p])

    def fire_w(u, p):
        k = u // NCHUNK
        j = u - k * NCHUNK
        base = wid * EW + j * C
        pltpu.async_copy(bufs[p], out_hbm.at[k, pl.ds(base, C)], wsem[p])

    def wait_g(p):
        pltpu.make_async_copy(x_hbm.at[idx_v.at[0]], bufs[p], gsem[p]).wait()

    def wait_w(p):
        pltpu.make_async_copy(bufs[p], out_hbm.at[0, pl.ds(0, C)],
                              wsem[p]).wait()

    for r in range(NBUF - 1):
        fire_g(r, r)

    def outer(uu, carry):
        for p in range(NBUF):
            u = NBUF * uu + p
            nxt = u + NBUF - 1
            pnxt = (p + NBUF - 1) % NBUF

            @pl.when(nxt < NUNIT)
            def _():
                @pl.when(u >= 1)
                def _():
                    wait_w(pnxt)
                fire_g(nxt, pnxt)

            wait_g(p)
            fire_w(u, p)
        return carry

    lax.fori_loop(0, NUNIT // NBUF, outer, 0)
    for p in range(NBUF):
        wait_w(p)


NBUF_S = 2


@functools.partial(
    pl.kernel,
    out_type=jax.ShapeDtypeStruct((NC, NPAD, F), jnp.float32),
    mesh=_mesh,
    scratch_types=(
        [pltpu.VMEM((NCHUNK, C), jnp.int32)]
        + [pltpu.VMEM((C, F), jnp.float32) for _ in range(NBUF_S)]
        + [pltpu.SemaphoreType.DMA for _ in range(NBUF_S)]
        + [pltpu.VMEM_SHARED((NPAD, F), jnp.float32)]
    ),
)
def _sc_scatter(e_hbm, dst3_hbm, zeros_hbm, out_hbm, idx_d, *rest):
    bufs = rest[:NBUF_S]
    lsem = rest[NBUF_S:2 * NBUF_S]
    shared = rest[2 * NBUF_S]
    cid = lax.axis_index("c")
    sid = lax.axis_index("s")
    wid = sid * NC + cid
    tbase = sid * ROWS_PER_TILE
    pltpu.sync_copy(zeros_hbm.at[pl.ds(tbase, ROWS_PER_TILE)],
                    shared.at[pl.ds(tbase, ROWS_PER_TILE)])
    pltpu.sync_copy(dst3_hbm.at[wid], idx_d)
    plsc.subcore_barrier()

    def fire_l(j, p):
        pltpu.async_copy(e_hbm.at[pl.ds(wid * EW + j * C, C)], bufs[p],
                         lsem[p])

    def wait_l(p):
        pltpu.make_async_copy(e_hbm.at[pl.ds(0, C)], bufs[p], lsem[p]).wait()

    def scat(j, p):
        wait_l(p)
        pltpu.sync_copy(bufs[p], shared.at[idx_d.at[j]], add=True)

    for r in range(NBUF_S):
        fire_l(r, r)

    def outer(jj, carry):
        for p in range(NBUF_S):
            j = NBUF_S * jj + p
            scat(j, p)

            @pl.when(j + NBUF_S < NCHUNK)
            def _():
                fire_l(j + NBUF_S, p)
        return carry

    lax.fori_loop(0, NCHUNK // NBUF_S, outer, 0)
    scat(NCHUNK - 1, (NCHUNK - 1) % NBUF_S)
    plsc.subcore_barrier()
    pltpu.sync_copy(shared.at[pl.ds(tbase, ROWS_PER_TILE)],
                    out_hbm.at[cid, pl.ds(tbase, ROWS_PER_TILE)])


def _mlp_body(nx, residual_idx, *refs):
    x_refs = refs[:nx]
    w1_refs = refs[nx:2 * nx]
    b1_ref, w2_ref, b2_ref, o_ref = refs[2 * nx:]

    def xval(i):
        r = x_refs[i]
        return r[0] if len(r.shape) == 3 else r[...]

    acc = b1_ref[0, :].astype(jnp.float32)
    acc = jnp.zeros_like(o_ref[...]) + acc[None, :]
    for i in range(nx):
        acc = acc + jnp.dot(xval(i), w1_refs[i][...],
                            preferred_element_type=jnp.float32)
    h = jax.nn.relu(acc)
    out = jnp.dot(h, w2_ref[...], preferred_element_type=jnp.float32)
    out = out + b2_ref[0, :][None, :]
    if residual_idx is not None:
        out = out + xval(residual_idx)
    o_ref[...] = out


def _x_spec(x, bm):
    if isinstance(x, tuple):
        arr, k = x
        return arr, pl.BlockSpec((1, bm, arr.shape[2]),
                                 lambda i, _k=k: (_k, i, 0))
    return x, pl.BlockSpec((bm, x.shape[1]), lambda i: (i, 0))


def _mlp(xs, w1s, b1, w2, b2, residual_idx=None, bm=None):
    nx = len(xs)
    m0 = xs[0][0].shape[1] if isinstance(xs[0], tuple) else xs[0].shape[0]
    if bm is None:
        bm = 8000 if m0 % 8000 == 0 else 2000
    arrs, xspecs = zip(*[_x_spec(x, bm) for x in xs])
    m = arrs[0].shape[1] if isinstance(xs[0], tuple) else arrs[0].shape[0]
    h_dim = w2.shape[0]
    o_dim = w2.shape[1]
    grid = (m // bm,)
    in_specs = (
        list(xspecs)
        + [pl.BlockSpec(w.shape, lambda i, _n=len(w.shape): (0,) * _n)
           for w in w1s]
        + [pl.BlockSpec((1, h_dim), lambda i: (0, 0)),
           pl.BlockSpec((h_dim, o_dim), lambda i: (0, 0)),
           pl.BlockSpec((1, o_dim), lambda i: (0, 0))]
    )
    return pl.pallas_call(
        functools.partial(_mlp_body, nx, residual_idx),
        grid=grid,
        in_specs=in_specs,
        out_specs=pl.BlockSpec((bm, o_dim), lambda i: (i, 0)),
        out_shape=jax.ShapeDtypeStruct((m, o_dim), jnp.float32),
    )(*arrs, *w1s, b1.reshape(1, -1), w2, b2.reshape(1, -1))


def kernel(x, edge_index, edge_attr,
           enc_nW1, enc_nb1, enc_nW2, enc_nb2,
           enc_eW1, enc_eb1, enc_eW2, enc_eb2,
           core_eW1, core_eb1, core_eW2, core_eb2,
           core_nW1, core_nb1, core_nW2, core_nb2,
           dec_nW1, dec_nb1, dec_nW2, dec_nb2,
           dec_eW1, dec_eb1, dec_eW2, dec_eb2):
    x = x.astype(jnp.float32)
    src4 = edge_index[0].reshape(NHALF, NW, NCHUNK, C)
    dst4 = edge_index[1].reshape(NHALF, NW, NCHUNK, C)
    idx2 = jnp.concatenate([src4, dst4], axis=2)
    e3 = edge_attr.astype(jnp.float32).reshape(NHALF, EH, F)
    zeros = jnp.zeros((NPAD, F), jnp.float32)

    x = _mlp([x], [enc_nW1], enc_nb1, enc_nW2, enc_nb2)
    eh = [_mlp([(e3, h)], [enc_eW1], enc_eb1, enc_eW2, enc_eb2)
          for h in range(NHALF)]

    eW1a = core_eW1[:F]
    eW1b = core_eW1[F:2 * F]
    eW1c = core_eW1[2 * F:]
    nW1a = core_nW1[:F]
    nW1b = core_nW1[F:]

    for _ in range(3):
        parts = []
        for h in range(NHALF):
            g = _sc_gather(x, idx2[h])
            eh[h] = _mlp([(g, 0), (g, 1), eh[h]], [eW1a, eW1b, eW1c],
                         core_eb1, core_eW2, core_eb2, residual_idx=2)
            parts.append(_sc_scatter(eh[h], dst4[h], zeros))
        agg_in = [(p, k) for p in parts for k in range(NC)]
        x = _mlp([x] + agg_in, [nW1a] + [nW1b] * (NHALF * NC), core_nb1,
                 core_nW2, core_nb2, residual_idx=0)

    x = _mlp([x], [dec_nW1], dec_nb1, dec_nW2, dec_nb2)
    e_out = _decode_e(eh[0], eh[1], dec_eW1, dec_eb1, dec_eW2, dec_eb2)
    return (x, e_out)


def _decode_body(a_ref, b_ref, w1_ref, b1_ref, w2_ref, b2_ref, o_ref):
    nblk = pl.num_programs(0) // 2
    first = pl.program_id(0) < nblk
    xin = jnp.where(first, a_ref[...], b_ref[...])
    acc = jnp.dot(xin, w1_ref[...], preferred_element_type=jnp.float32)
    h = jax.nn.relu(acc + b1_ref[0, :][None, :])
    out = jnp.dot(h, w2_ref[...], preferred_element_type=jnp.float32)
    o_ref[...] = out + b2_ref[0, :][None, :]


def _decode_e(ea, eb, w1, b1, w2, b2, bm=8000):
    nblk = EH // bm
    h_dim = w2.shape[0]
    o_dim = w2.shape[1]
    in_specs = [
        pl.BlockSpec((bm, ea.shape[1]),
                     lambda i: (jnp.where(i < nblk, i, nblk - 1), 0)),
        pl.BlockSpec((bm, eb.shape[1]),
                     lambda i: (jnp.where(i < nblk, 0, i - nblk), 0)),
        pl.BlockSpec(w1.shape, lambda i: (0, 0)),
        pl.BlockSpec((1, h_dim), lambda i: (0, 0)),
        pl.BlockSpec((h_dim, o_dim), lambda i: (0, 0)),
        pl.BlockSpec((1, o_dim), lambda i: (0, 0)),
    ]
    return pl.pallas_call(
        _decode_body,
        grid=(2 * nblk,),
        in_specs=in_specs,
        out_specs=pl.BlockSpec((bm, o_dim), lambda i: (i, 0)),
        out_shape=jax.ShapeDtypeStruct((E, o_dim), jnp.float32),
    )(ea, eb, w1, b1.reshape(1, -1), w2, b2.reshape(1, -1))


# speedup vs baseline: 1.0063x; 1.0063x over previous
import functools

import jax
import jax.numpy as jnp
from jax import lax
from jax.experimental import pallas as pl
from jax.experimental.pallas import tpu as pltpu
from jax.experimental.pallas import tpu_sc as plsc

N = 10000
E = 320000
F = 128
NHALF = 2
EH = E // NHALF

NC = 2
NS = 16
NW = NC * NS

EW = EH // NW
C = 40
NCHUNK = EW // C
NUNIT = 2 * NCHUNK
NBUF = 5
assert NUNIT % NBUF == 0
NPAD = 10240
ROWS_PER_TILE = NPAD // NS

_mesh = plsc.VectorSubcoreMesh(core_axis_name="c", subcore_axis_name="s",
                               num_cores=NC, num_subcores=NS)


@functools.partial(
    pl.kernel,
    out_type=jax.ShapeDtypeStruct((2, EH, F), jnp.float32),
    mesh=_mesh,
    scratch_types=(
        [pltpu.VMEM((NUNIT, C), jnp.int32)]
        + [pltpu.VMEM((C, F), jnp.float32) for _ in range(NBUF)]
        + [pltpu.SemaphoreType.DMA for _ in range(2 * NBUF)]
    ),
)
def _sc_gather(x_hbm, idx2_hbm, out_hbm, idx_v, *bufs_and_sems):
    bufs = bufs_and_sems[:NBUF]
    gsem = bufs_and_sems[NBUF:2 * NBUF]
    wsem = bufs_and_sems[2 * NBUF:]
    wid = lax.axis_index("s") * NC + lax.axis_index("c")
    pltpu.sync_copy(idx2_hbm.at[wid], idx_v)

    def fire_g(u, p):
        pltpu.async_copy(x_hbm.at[idx_v.at[u]], bufs[p], gsem[p])

    def fire_w(u, p):
        k = u // NCHUNK
        j = u - k * NCHUNK
        base = wid * EW + j * C
        pltpu.async_copy(bufs[p], out_hbm.at[k, pl.ds(base, C)], wsem[p])

    def wait_g(p):
        pltpu.make_async_copy(x_hbm.at[idx_v.at[0]], bufs[p], gsem[p]).wait()

    def wait_w(p):
        pltpu.make_async_copy(bufs[p], out_hbm.at[0, pl.ds(0, C)],
                              wsem[p]).wait()

    for r in range(NBUF - 1):
        fire_g(r, r)

    def outer(uu, carry):
        for p in range(NBUF):
            u = NBUF * uu + p
            nxt = u + NBUF - 1
            pnxt = (p + NBUF - 1) % NBUF

            @pl.when(nxt < NUNIT)
            def _():
                @pl.when(u >= 1)
                def _():
                    wait_w(pnxt)
                fire_g(nxt, pnxt)

            wait_g(p)
            fire_w(u, p)
        return carry

    lax.fori_loop(0, NUNIT // NBUF, outer, 0)
    for p in range(NBUF):
        wait_w(p)


NBUF_S = 2


@functools.partial(
    pl.kernel,
    out_type=jax.ShapeDtypeStruct((NC, NPAD, F), jnp.float32),
    mesh=_mesh,
    scratch_types=(
        [pltpu.VMEM((NCHUNK, C), jnp.int32)]
        + [pltpu.VMEM((C, F), jnp.float32) for _ in range(NBUF_S)]
        + [pltpu.SemaphoreType.DMA for _ in range(NBUF_S)]
        + [pltpu.VMEM_SHARED((NPAD, F), jnp.float32)]
    ),
)
def _sc_scatter(e_hbm, dst3_hbm, zeros_hbm, out_hbm, idx_d, *rest):
    bufs = rest[:NBUF_S]
    lsem = rest[NBUF_S:2 * NBUF_S]
    shared = rest[2 * NBUF_S]
    cid = lax.axis_index("c")
    sid = lax.axis_index("s")
    wid = sid * NC + cid
    tbase = sid * ROWS_PER_TILE
    pltpu.sync_copy(zeros_hbm.at[pl.ds(tbase, ROWS_PER_TILE)],
                    shared.at[pl.ds(tbase, ROWS_PER_TILE)])
    pltpu.sync_copy(dst3_hbm.at[wid], idx_d)
    plsc.subcore_barrier()

    def fire_l(j, p):
        pltpu.async_copy(e_hbm.at[pl.ds(wid * EW + j * C, C)], bufs[p],
                         lsem[p])

    def wait_l(p):
        pltpu.make_async_copy(e_hbm.at[pl.ds(0, C)], bufs[p], lsem[p]).wait()

    def scat(j, p):
        wait_l(p)
        pltpu.sync_copy(bufs[p], shared.at[idx_d.at[j]], add=True)

    for r in range(NBUF_S):
        fire_l(r, r)

    def outer(jj, carry):
        for p in range(NBUF_S):
            j = NBUF_S * jj + p
            scat(j, p)

            @pl.when(j + NBUF_S < NCHUNK)
            def _():
                fire_l(j + NBUF_S, p)
        return carry

    lax.fori_loop(0, NCHUNK // NBUF_S, outer, 0)
    scat(NCHUNK - 1, (NCHUNK - 1) % NBUF_S)
    plsc.subcore_barrier()
    pltpu.sync_copy(shared.at[pl.ds(tbase, ROWS_PER_TILE)],
                    out_hbm.at[cid, pl.ds(tbase, ROWS_PER_TILE)])


def _mlp_body(nx, residual_idx, *refs):
    x_refs = refs[:nx]
    w1_refs = refs[nx:2 * nx]
    b1_ref, w2_ref, b2_ref, o_ref = refs[2 * nx:]

    def xval(i):
        r = x_refs[i]
        return r[0] if len(r.shape) == 3 else r[...]

    acc = b1_ref[0, :].astype(jnp.float32)
    acc = jnp.zeros_like(o_ref[...]) + acc[None, :]
    for i in range(nx):
        acc = acc + jnp.dot(xval(i), w1_refs[i][...],
                            preferred_element_type=jnp.float32)
    h = jax.nn.relu(acc)
    out = jnp.dot(h, w2_ref[...], preferred_element_type=jnp.float32)
    out = out + b2_ref[0, :][None, :]
    if residual_idx is not None:
        out = out + xval(residual_idx)
    o_ref[...] = out


def _x_spec(x, bm):
    if isinstance(x, tuple):
        arr, k = x
        return arr, pl.BlockSpec((1, bm, arr.shape[2]),
                                 lambda i, _k=k: (_k, i, 0))
    return x, pl.BlockSpec((bm, x.shape[1]), lambda i: (i, 0))


def _mlp(xs, w1s, b1, w2, b2, residual_idx=None, bm=None):
    nx = len(xs)
    m0 = xs[0][0].shape[1] if isinstance(xs[0], tuple) else xs[0].shape[0]
    if bm is None:
        bm = 8000 if m0 % 8000 == 0 else 2000
    arrs, xspecs = zip(*[_x_spec(x, bm) for x in xs])
    m = arrs[0].shape[1] if isinstance(xs[0], tuple) else arrs[0].shape[0]
    h_dim = w2.shape[0]
    o_dim = w2.shape[1]
    grid = (m // bm,)
    in_specs = (
        list(xspecs)
        + [pl.BlockSpec(w.shape, lambda i, _n=len(w.shape): (0,) * _n)
           for w in w1s]
        + [pl.BlockSpec((1, h_dim), lambda i: (0, 0)),
           pl.BlockSpec((h_dim, o_dim), lambda i: (0, 0)),
           pl.BlockSpec((1, o_dim), lambda i: (0, 0))]
    )
    return pl.pallas_call(
        functools.partial(_mlp_body, nx, residual_idx),
        grid=grid,
        in_specs=in_specs,
        out_specs=pl.BlockSpec((bm, o_dim), lambda i: (i, 0)),
        out_shape=jax.ShapeDtypeStruct((m, o_dim), jnp.float32),
    )(*arrs, *w1s, b1.reshape(1, -1), w2, b2.reshape(1, -1))


def kernel(x, edge_index, edge_attr,
           enc_nW1, enc_nb1, enc_nW2, enc_nb2,
           enc_eW1, enc_eb1, enc_eW2, enc_eb2,
           core_eW1, core_eb1, core_eW2, core_eb2,
           core_nW1, core_nb1, core_nW2, core_nb2,
           dec_nW1, dec_nb1, dec_nW2, dec_nb2,
           dec_eW1, dec_eb1, dec_eW2, dec_eb2):
    x = x.astype(jnp.float32)
    src4 = edge_index[0].reshape(NHALF, NW, NCHUNK, C)
    dst4 = edge_index[1].reshape(NHALF, NW, NCHUNK, C)
    idx2 = jnp.concatenate([src4, dst4], axis=2)
    e3 = edge_attr.astype(jnp.float32).reshape(NHALF, EH, F)
    zeros = jnp.zeros((NPAD, F), jnp.float32)

    x = _mlp([x], [enc_nW1], enc_nb1, enc_nW2, enc_nb2)
    eh = [_mlp([(e3, h)], [enc_eW1], enc_eb1, enc_eW2, enc_eb2)
          for h in range(NHALF)]

    eW1a = core_eW1[:F]
    eW1b = core_eW1[F:2 * F]
    eW1c = core_eW1[2 * F:]
    nW1a = core_nW1[:F]
    nW1b = core_nW1[F:]

    for _ in range(3):
        parts = []
        for h in range(NHALF):
            g = _sc_gather(x, idx2[h])
            eh[h] = _mlp([(g, 0), (g, 1), eh[h]], [eW1a, eW1b, eW1c],
                         core_eb1, core_eW2, core_eb2, residual_idx=2)
            parts.append(_sc_scatter(eh[h], dst4[h], zeros))
        agg_in = [(p, k) for p in parts for k in range(NC)]
        x = _mlp([x] + agg_in, [nW1a] + [nW1b] * (NHALF * NC), core_nb1,
                 core_nW2, core_nb2, residual_idx=0)

    x = _mlp([x], [dec_nW1], dec_nb1, dec_nW2, dec_nb2)
    e_out = _decode_e(eh[0], eh[1], dec_eW1, dec_eb1, dec_eW2, dec_eb2)
    return (x, e_out)


def _decode_body(a_ref, b_ref, w1_ref, b1_ref, w2_ref, b2_ref, o_ref):
    nblk = pl.num_programs(0) // 2
    first = pl.program_id(0) < nblk
    xin = jnp.where(first, a_ref[...], b_ref[...])
    acc = jnp.dot(xin, w1_ref[...], preferred_element_type=jnp.float32)
    h = jax.nn.relu(acc + b1_ref[0, :][None, :])
    out = jnp.dot(h, w2_ref[...], preferred_element_type=jnp.float32)
    o_ref[...] = out + b2_ref[0, :][None, :]


def _decode_e(ea, eb, w1, b1, w2, b2, bm=4000):
    nblk = EH // bm
    h_dim = w2.shape[0]
    o_dim = w2.shape[1]
    in_specs = [
        pl.BlockSpec((bm, ea.shape[1]),
                     lambda i: (jnp.where(i < nblk, i, nblk - 1), 0)),
        pl.BlockSpec((bm, eb.shape[1]),
                     lambda i: (jnp.where(i < nblk, 0, i - nblk), 0)),
        pl.BlockSpec(w1.shape, lambda i: (0, 0)),
        pl.BlockSpec((1, h_dim), lambda i: (0, 0)),
        pl.BlockSpec((h_dim, o_dim), lambda i: (0, 0)),
        pl.BlockSpec((1, o_dim), lambda i: (0, 0)),
    ]
    return pl.pallas_call(
        _decode_body,
        grid=(2 * nblk,),
        in_specs=in_specs,
        out_specs=pl.BlockSpec((bm, o_dim), lambda i: (i, 0)),
        out_shape=jax.ShapeDtypeStruct((E, o_dim), jnp.float32),
    )(ea, eb, w1, b1.reshape(1, -1), w2, b2.reshape(1, -1))
